# baseline (speedup 1.0000x reference)
import functools

import jax
import jax.numpy as jnp
from jax import lax
from jax.experimental import pallas as pl
from jax.experimental.pallas import tpu as pltpu
from jax.experimental.pallas import tpu_sc as plsc

NC = 2
NS = 16
NW = NC * NS
B_E = 128
CW = 32
NBUF = 4

F32 = jnp.float32


def _sc_mesh():
    return plsc.VectorSubcoreMesh(core_axis_name="c", subcore_axis_name="s")


_SC_PARAMS = pltpu.CompilerParams(use_tc_tiling_on_sc=False)


@functools.partial(jax.jit, static_argnames=("nb", "n_pad"))
def _deg_sc(dst3, nb, n_pad):
    rpt = n_pad // NS

    @functools.partial(
        pl.kernel,
        out_type=jax.ShapeDtypeStruct((NC, n_pad, 16), F32),
        mesh=_sc_mesh(),
        scratch_types=[
            pltpu.VMEM((nb, B_E), jnp.int32),
            pltpu.VMEM((B_E, 16), F32),
            pltpu.VMEM_SHARED((n_pad, 16), F32),
        ],
        compiler_params=_SC_PARAMS,
    )
    def k(dst_hbm, out_hbm, idx_v, ones_v, acc_sh):
        c = lax.axis_index("c")
        s = lax.axis_index("s")
        wid = c * NS + s

        def fill(i, val):
            ones_v[i, :] = jnp.full((16,), val, F32)
            return val

        lax.fori_loop(0, B_E, fill, 0.0)
        for kk in range(rpt // B_E):
            pltpu.sync_copy(ones_v, acc_sh.at[pl.ds(s * rpt + kk * B_E, B_E)])
        lax.fori_loop(0, B_E, fill, 1.0)
        pltpu.sync_copy(dst_hbm.at[wid], idx_v)
        plsc.subcore_barrier()

        def body(j, carry):
            pltpu.sync_copy(ones_v, acc_sh.at[idx_v.at[j]], add=True)
            return carry

        lax.fori_loop(0, nb, body, 0)
        plsc.subcore_barrier()
        pltpu.sync_copy(
            acc_sh.at[pl.ds(s * rpt, rpt)],
            out_hbm.at[c, pl.ds(s * rpt, rpt)],
        )

    return k(dst3)


@functools.partial(jax.jit, static_argnames=("nb", "n_pad", "n_chunks"))
def _spmm_sc(xs, src2, dst2, nb, n_pad, n_chunks):
    n = xs.shape[1]
    rpt = n_pad // NS
    rows_ps = n // NS
    rows_rem = n - rows_ps * NS
    nch2 = n_chunks // NC

    @functools.partial(
        pl.kernel,
        out_type=jax.ShapeDtypeStruct((n_chunks, n_pad, CW), F32),
        mesh=_sc_mesh(),
        scratch_types=[
            pltpu.VMEM((nb, B_E), jnp.int32),
            pltpu.VMEM((nb, B_E), jnp.int32),
            pltpu.VMEM((NBUF, B_E, CW), F32),
            pltpu.VMEM((B_E, CW), F32),
            pltpu.VMEM_SHARED((n, CW), F32),
            pltpu.VMEM_SHARED((n_pad, CW), F32),
        ] + [pltpu.SemaphoreType.DMA] * NBUF,
        compiler_params=_SC_PARAMS,
    )
    def k(xs_hbm, src_hbm, dst_hbm, out_hbm, src_v, dst_v, rows_v,
          zeros_v, xs_sh, acc_sh, *gsem):
        c = lax.axis_index("c")
        s = lax.axis_index("s")

        def zfill(i, carry):
            for kk in range(CW // 16):
                zeros_v[i, pl.ds(kk * 16, 16)] = jnp.zeros((16,), F32)
            return carry

        lax.fori_loop(0, B_E, zfill, 0)
        pltpu.sync_copy(src_hbm.at[s], src_v)
        pltpu.sync_copy(dst_hbm.at[s], dst_v)

        def gath(j, b):
            return pltpu.make_async_copy(
                xs_sh.at[src_v.at[j]], rows_v.at[b], gsem[b])

        for cl in range(nch2):
            ci = c * nch2 + cl
            for kk in range(rpt // B_E):
                base = s * rpt + kk * B_E
                pltpu.sync_copy(zeros_v, acc_sh.at[pl.ds(base, B_E)])
            pltpu.sync_copy(
                xs_hbm.at[ci].at[pl.ds(s * rows_ps, rows_ps)],
                xs_sh.at[pl.ds(s * rows_ps, rows_ps)],
            )
            if rows_rem:
                @pl.when(s == NS - 1)
                def _():
                    pltpu.sync_copy(
                        xs_hbm.at[ci].at[pl.ds(NS * rows_ps, rows_rem)],
                        xs_sh.at[pl.ds(NS * rows_ps, rows_rem)],
                    )
            plsc.subcore_barrier()
            for b in range(NBUF):
                gath(b, b).start()

            def body(jj, carry):
                for b in range(NBUF):
                    j = jj * NBUF + b
                    gath(j, b).wait()
                    pltpu.sync_copy(rows_v.at[b], acc_sh.at[dst_v.at[j]],
                                    add=True)

                    @pl.when(jj + 1 < nb // NBUF)
                    def _():
                        gath(j + NBUF, b).start()
                return carry

            lax.fori_loop(0, nb // NBUF, body, 0)
            plsc.subcore_barrier()
            pltpu.sync_copy(
                acc_sh.at[pl.ds(s * rpt, rpt)],
                out_hbm.at[ci].at[pl.ds(s * rpt, rpt)],
            )

    return k(xs, src2, dst2)


def _dinv_from(degp_blk):
    deg = degp_blk[0, :, 0] + degp_blk[1, :, 0]
    return jnp.where(deg > 0.0, lax.rsqrt(deg), 0.0)


def _prep_tc(x, degp):
    n, f = x.shape
    bn = 400
    c_out = f // CW

    def body(x_ref, degp_ref, xs_ref):
        dinv = _dinv_from(degp_ref)
        xs = x_ref[...] * dinv[:, None]
        for co in range(c_out):
            xs_ref[co] = xs[:, co * CW:(co + 1) * CW]

    return pl.pallas_call(
        body,
        grid=(n // bn,),
        in_specs=[
            pl.BlockSpec((bn, f), lambda i: (i, 0)),
            pl.BlockSpec((2, bn, 16), lambda i: (0, i, 0)),
        ],
        out_specs=pl.BlockSpec((c_out, bn, CW), lambda i: (0, i, 0)),
        out_shape=jax.ShapeDtypeStruct((c_out, n, CW), F32),
    )(x, degp)


def _mm0_tc(h, w0, b):
    n, f_in = h.shape
    f_out = w0.shape[1]
    bn = 400
    b2 = b.reshape(1, f_out)

    def body(h_ref, w0_ref, b_ref, o_ref):
        o_ref[...] = jnp.dot(h_ref[...], w0_ref[...],
                             preferred_element_type=F32) + b_ref[...]

    return pl.pallas_call(
        body,
        grid=(n // bn,),
        in_specs=[
            pl.BlockSpec((bn, f_in), lambda i: (i, 0)),
            pl.BlockSpec((f_in, f_out), lambda i: (0, 0)),
            pl.BlockSpec((1, f_out), lambda i: (0, 0)),
        ],
        out_specs=pl.BlockSpec((bn, f_out), lambda i: (i, 0)),
        out_shape=jax.ShapeDtypeStruct((n, f_out), F32),
    )(h, w0, b2)


def _combine_tc(p0, tp, degp, w1, last):
    n, f_out = p0.shape
    f_in = w1.shape[0]
    c_in = f_in // CW
    bn = 400

    def body(p0_ref, tp_ref, degp_ref, w1_ref, *out_refs):
        dinv = _dinv_from(degp_ref)
        mdinv = -dinv
        t = jnp.concatenate(
            [tp_ref[ci] * mdinv[:, None] for ci in range(c_in)], axis=1)
        acc = p0_ref[...] + jnp.dot(t, w1_ref[...],
                                    preferred_element_type=F32)
        hn = jnp.maximum(acc, 0.0)
        out_refs[0][...] = hn
        if not last:
            dcol = dinv[:, None]
            for co in range(f_out // CW):
                out_refs[1][co] = hn[:, co * CW:(co + 1) * CW] * dcol

    out_shape = [jax.ShapeDtypeStruct((n, f_out), F32)]
    out_specs = [pl.BlockSpec((bn, f_out), lambda i: (i, 0))]
    if not last:
        out_shape.append(jax.ShapeDtypeStruct((f_out // CW, n, CW), F32))
        out_specs.append(
            pl.BlockSpec((f_out // CW, bn, CW), lambda i: (0, i, 0)))

    return pl.pallas_call(
        body,
        grid=(n // bn,),
        in_specs=[
            pl.BlockSpec((bn, f_out), lambda i: (i, 0)),
            pl.BlockSpec((c_in, bn, CW), lambda i: (0, i, 0)),
            pl.BlockSpec((2, bn, 16), lambda i: (0, i, 0)),
            pl.BlockSpec((f_in, f_out), lambda i: (0, 0)),
        ],
        out_specs=out_specs,
        out_shape=out_shape,
    )(p0, tp, degp, w1)


def kernel(x, edge_index, W0_1, W1_1, b_1, W0_2, W1_2, b_2, W0_3, W1_3, b_3):
    n = x.shape[0]
    e = edge_index.shape[1]

    e_pad = -(-e // (NS * B_E * NBUF)) * (NS * B_E * NBUF)
    nb = e_pad // (NS * B_E)
    nb_deg = e_pad // (NW * B_E)
    pad = e_pad - e
    n_pad = -(-n // (NS * B_E)) * (NS * B_E)

    src = jnp.concatenate([edge_index[0], jnp.zeros((pad,), jnp.int32)])
    dst = jnp.concatenate([edge_index[1], jnp.full((pad,), n, jnp.int32)])
    src2 = src.reshape(NS, nb, B_E)
    dst2 = dst.reshape(NS, nb, B_E)

    degp = _deg_sc(dst.reshape(NW, nb_deg, B_E), nb=nb_deg, n_pad=n_pad)

    xs = _prep_tc(x, degp)
    h = x
    params = [(W0_1, W1_1, b_1), (W0_2, W1_2, b_2), (W0_3, W1_3, b_3)]
    for li, (w0, w1, b) in enumerate(params):
        tp = _spmm_sc(xs, src2, dst2, nb=nb, n_pad=n_pad,
                      n_chunks=h.shape[1] // CW)
        p0 = _mm0_tc(h, w0, b)
        last = li == 2
        outs = _combine_tc(p0, tp, degp, w1, last)
        if last:
            h = outs[0]
        else:
            h, xs = outs
    return h


# speedup vs baseline: 1.0150x; 1.0150x over previous
import functools

import jax
import jax.numpy as jnp
from jax import lax
from jax.experimental import pallas as pl
from jax.experimental.pallas import tpu as pltpu
from jax.experimental.pallas import tpu_sc as plsc

NC = 2
NS = 16
NW = NC * NS
B_E = 128
CW = 32
NBUF = 4

F32 = jnp.float32


def _sc_mesh():
    return plsc.VectorSubcoreMesh(core_axis_name="c", subcore_axis_name="s")


_SC_PARAMS = pltpu.CompilerParams(use_tc_tiling_on_sc=False)


@functools.partial(jax.jit, static_argnames=("nb", "n_pad"))
def _deg_sc(dst3, nb, n_pad):
    rpt = n_pad // NS

    @functools.partial(
        pl.kernel,
        out_type=jax.ShapeDtypeStruct((NC, n_pad, 16), F32),
        mesh=_sc_mesh(),
        scratch_types=[
            pltpu.VMEM((nb, B_E), jnp.int32),
            pltpu.VMEM((B_E, 16), F32),
            pltpu.VMEM_SHARED((n_pad, 16), F32),
        ],
        compiler_params=_SC_PARAMS,
    )
    def k(dst_hbm, out_hbm, idx_v, ones_v, acc_sh):
        c = lax.axis_index("c")
        s = lax.axis_index("s")
        wid = c * NS + s

        def fill(i, val):
            ones_v[i, :] = jnp.full((16,), val, F32)
            return val

        lax.fori_loop(0, B_E, fill, 0.0)
        for kk in range(rpt // B_E):
            pltpu.sync_copy(ones_v, acc_sh.at[pl.ds(s * rpt + kk * B_E, B_E)])
        lax.fori_loop(0, B_E, fill, 1.0)
        pltpu.sync_copy(dst_hbm.at[wid], idx_v)
        plsc.subcore_barrier()

        def body(j, carry):
            pltpu.sync_copy(ones_v, acc_sh.at[idx_v.at[j]], add=True)
            return carry

        lax.fori_loop(0, nb, body, 0)
        plsc.subcore_barrier()
        pltpu.sync_copy(
            acc_sh.at[pl.ds(s * rpt, rpt)],
            out_hbm.at[c, pl.ds(s * rpt, rpt)],
        )

    return k(dst3)


@functools.partial(jax.jit, static_argnames=("nb", "n_pad", "n_chunks"))
def _spmm_sc(xs, src2, dst2, nb, n_pad, n_chunks):
    n = xs.shape[1]
    rpt = n_pad // NS
    rows_ps = n // NS
    rows_rem = n - rows_ps * NS
    nch2 = n_chunks // NC

    @functools.partial(
        pl.kernel,
        out_type=jax.ShapeDtypeStruct((n_chunks, n_pad, CW), F32),
        mesh=_sc_mesh(),
        scratch_types=[
            pltpu.VMEM((nb, B_E), jnp.int32),
            pltpu.VMEM((nb, B_E), jnp.int32),
            pltpu.VMEM((NBUF, B_E, CW), F32),
            pltpu.VMEM((B_E, CW), F32),
            pltpu.VMEM_SHARED((n, CW), F32),
            pltpu.VMEM_SHARED((n_pad, CW), F32),
        ] + [pltpu.SemaphoreType.DMA] * NBUF,
        compiler_params=_SC_PARAMS,
    )
    def k(xs_hbm, src_hbm, dst_hbm, out_hbm, src_v, dst_v, rows_v,
          zeros_v, xs_sh, acc_sh, *gsem):
        c = lax.axis_index("c")
        s = lax.axis_index("s")

        def zfill(i, carry):
            for kk in range(CW // 16):
                zeros_v[i, pl.ds(kk * 16, 16)] = jnp.zeros((16,), F32)
            return carry

        lax.fori_loop(0, B_E, zfill, 0)
        pltpu.sync_copy(src_hbm.at[s], src_v)
        pltpu.sync_copy(dst_hbm.at[s], dst_v)

        def gath(j, b):
            return pltpu.make_async_copy(
                xs_sh.at[src_v.at[j]], rows_v.at[b], gsem[b])

        for cl in range(nch2):
            ci = c * nch2 + cl
            for kk in range(rpt // B_E):
                base = s * rpt + kk * B_E
                pltpu.sync_copy(zeros_v, acc_sh.at[pl.ds(base, B_E)])
            pltpu.sync_copy(
                xs_hbm.at[ci].at[pl.ds(s * rows_ps, rows_ps)],
                xs_sh.at[pl.ds(s * rows_ps, rows_ps)],
            )
            if rows_rem:
                @pl.when(s == NS - 1)
                def _():
                    pltpu.sync_copy(
                        xs_hbm.at[ci].at[pl.ds(NS * rows_ps, rows_rem)],
                        xs_sh.at[pl.ds(NS * rows_ps, rows_rem)],
                    )
            plsc.subcore_barrier()
            for b in range(NBUF):
                gath(b, b).start()

            def body(jj, carry):
                for b in range(NBUF):
                    j = jj * NBUF + b
                    gath(j, b).wait()
                    pltpu.sync_copy(rows_v.at[b], acc_sh.at[dst_v.at[j]],
                                    add=True)

                    @pl.when(jj + 1 < nb // NBUF)
                    def _():
                        gath(j + NBUF, b).start()
                return carry

            lax.fori_loop(0, nb // NBUF, body, 0)
            plsc.subcore_barrier()
            pltpu.sync_copy(
                acc_sh.at[pl.ds(s * rpt, rpt)],
                out_hbm.at[ci].at[pl.ds(s * rpt, rpt)],
            )

    return k(xs, src2, dst2)


def _dinv_from(degp_blk):
    deg = degp_blk[0, :, 0] + degp_blk[1, :, 0]
    return jnp.where(deg > 0.0, lax.rsqrt(deg), 0.0)


def _prep_tc(x, degp):
    n, f = x.shape
    bn = 1000
    c_out = f // CW

    def body(x_ref, degp_ref, xs_ref):
        dinv = _dinv_from(degp_ref)
        xs = x_ref[...] * dinv[:, None]
        for co in range(c_out):
            xs_ref[co] = xs[:, co * CW:(co + 1) * CW]

    return pl.pallas_call(
        body,
        grid=(n // bn,),
        in_specs=[
            pl.BlockSpec((bn, f), lambda i: (i, 0)),
            pl.BlockSpec((2, bn, 16), lambda i: (0, i, 0)),
        ],
        out_specs=pl.BlockSpec((c_out, bn, CW), lambda i: (0, i, 0)),
        out_shape=jax.ShapeDtypeStruct((c_out, n, CW), F32),
    )(x, degp)


def _mm0_tc(h, w0, b):
    n, f_in = h.shape
    f_out = w0.shape[1]
    bn = 1000
    b2 = b.reshape(1, f_out)

    def body(h_ref, w0_ref, b_ref, o_ref):
        o_ref[...] = jnp.dot(h_ref[...], w0_ref[...],
                             preferred_element_type=F32) + b_ref[...]

    return pl.pallas_call(
        body,
        grid=(n // bn,),
        in_specs=[
            pl.BlockSpec((bn, f_in), lambda i: (i, 0)),
            pl.BlockSpec((f_in, f_out), lambda i: (0, 0)),
            pl.BlockSpec((1, f_out), lambda i: (0, 0)),
        ],
        out_specs=pl.BlockSpec((bn, f_out), lambda i: (i, 0)),
        out_shape=jax.ShapeDtypeStruct((n, f_out), F32),
    )(h, w0, b2)


def _combine_tc(p0, tp, degp, w1, last):
    n, f_out = p0.shape
    f_in = w1.shape[0]
    c_in = f_in // CW
    bn = 1000

    def body(p0_ref, tp_ref, degp_ref, w1_ref, *out_refs):
        dinv = _dinv_from(degp_ref)
        mdinv = -dinv
        t = jnp.concatenate(
            [tp_ref[ci] * mdinv[:, None] for ci in range(c_in)], axis=1)
        acc = p0_ref[...] + jnp.dot(t, w1_ref[...],
                                    preferred_element_type=F32)
        hn = jnp.maximum(acc, 0.0)
        out_refs[0][...] = hn
        if not last:
            dcol = dinv[:, None]
            for co in range(f_out // CW):
                out_refs[1][co] = hn[:, co * CW:(co + 1) * CW] * dcol

    out_shape = [jax.ShapeDtypeStruct((n, f_out), F32)]
    out_specs = [pl.BlockSpec((bn, f_out), lambda i: (i, 0))]
    if not last:
        out_shape.append(jax.ShapeDtypeStruct((f_out // CW, n, CW), F32))
        out_specs.append(
            pl.BlockSpec((f_out // CW, bn, CW), lambda i: (0, i, 0)))

    return pl.pallas_call(
        body,
        grid=(n // bn,),
        in_specs=[
            pl.BlockSpec((bn, f_out), lambda i: (i, 0)),
            pl.BlockSpec((c_in, bn, CW), lambda i: (0, i, 0)),
            pl.BlockSpec((2, bn, 16), lambda i: (0, i, 0)),
            pl.BlockSpec((f_in, f_out), lambda i: (0, 0)),
        ],
        out_specs=out_specs,
        out_shape=out_shape,
    )(p0, tp, degp, w1)


def kernel(x, edge_index, W0_1, W1_1, b_1, W0_2, W1_2, b_2, W0_3, W1_3, b_3):
    n = x.shape[0]
    e = edge_index.shape[1]

    e_pad = -(-e // (NS * B_E * NBUF)) * (NS * B_E * NBUF)
    nb = e_pad // (NS * B_E)
    nb_deg = e_pad // (NW * B_E)
    pad = e_pad - e
    n_pad = -(-n // (NS * B_E)) * (NS * B_E)

    src = jnp.concatenate([edge_index[0], jnp.zeros((pad,), jnp.int32)])
    dst = jnp.concatenate([edge_index[1], jnp.full((pad,), n, jnp.int32)])
    src2 = src.reshape(NS, nb, B_E)
    dst2 = dst.reshape(NS, nb, B_E)

    degp = _deg_sc(dst.reshape(NW, nb_deg, B_E), nb=nb_deg, n_pad=n_pad)

    xs = _prep_tc(x, degp)
    h = x
    params = [(W0_1, W1_1, b_1), (W0_2, W1_2, b_2), (W0_3, W1_3, b_3)]
    for li, (w0, w1, b) in enumerate(params):
        tp = _spmm_sc(xs, src2, dst2, nb=nb, n_pad=n_pad,
                      n_chunks=h.shape[1] // CW)
        p0 = _mm0_tc(h, w0, b)
        last = li == 2
        outs = _combine_tc(p0, tp, degp, w1, last)
        if last:
            h = outs[0]
        else:
            h, xs = outs
    return h
